# baseline (speedup 1.0000x reference)
import functools
import math

import jax
import jax.numpy as jnp
from jax.experimental import pallas as pl
from jax.experimental.pallas import tpu as pltpu

EPS = 1e-5
NEG_BIG = -1e30


def _log_sigmoid_pair(x):
    z = jnp.exp(-jnp.abs(x))
    l1p = jnp.log1p(z)
    return jnp.minimum(x, 0.0) - l1p, jnp.minimum(-x, 0.0) - l1p


def _qkv_body(x_ref, ln_ref, wq_ref, wk_ref, wv_ref, q_ref, k_ref, v_ref, *, scale):
    x = x_ref[...]
    ms = jnp.mean(x * x, axis=-1, keepdims=True)
    xn = x * jax.lax.rsqrt(ms + EPS) * ln_ref[...]
    q_ref[...] = jnp.dot(xn, wq_ref[...], preferred_element_type=jnp.float32) * scale
    k_ref[...] = jnp.dot(xn, wk_ref[...], preferred_element_type=jnp.float32)
    v_ref[...] = jnp.dot(xn.astype(jnp.bfloat16), wv_ref[...],
                         preferred_element_type=jnp.float32).astype(jnp.bfloat16)


def _qkv(x, ln1_w, wq, wk, wv, dh, bt):
    s, d = x.shape
    grid = (s // bt,)
    spec_row = pl.BlockSpec((bt, d), lambda t: (t, 0))
    spec_w = pl.BlockSpec((d, d), lambda t: (0, 0))
    spec_ln = pl.BlockSpec((1, d), lambda t: (0, 0))
    out = jax.ShapeDtypeStruct((s, d), jnp.float32)
    out_bf = jax.ShapeDtypeStruct((s, d), jnp.bfloat16)
    return pl.pallas_call(
        functools.partial(_qkv_body, scale=1.0 / math.sqrt(dh)),
        grid=grid,
        in_specs=[spec_row, spec_ln, spec_w, spec_w, spec_w],
        out_specs=[spec_row, spec_row, spec_row],
        out_shape=[out, out, out_bf],
        compiler_params=pltpu.CompilerParams(dimension_semantics=("parallel",)),
    )(x, ln1_w, wq, wk, wv.astype(jnp.bfloat16))


def _suffix_cumsum_incl(x, bk, split):
    tri = (jax.lax.broadcasted_iota(jnp.int32, (bk, bk), 0)
           >= jax.lax.broadcasted_iota(jnp.int32, (bk, bk), 1)).astype(jnp.bfloat16)
    hi = x.astype(jnp.bfloat16)
    cs = jnp.dot(hi, tri, preferred_element_type=jnp.float32)
    if split:
        lo = (x - hi.astype(jnp.float32)).astype(jnp.bfloat16)
        cs = cs + jnp.dot(lo, tri, preferred_element_type=jnp.float32)
    return cs


def _attn_body(qi_ref, a_ref, q_ref, k_ref, v_ref, o_ref, carry_ref, *, bq, bk):
    t = pl.program_id(1)
    qi = qi_ref[t]
    a = a_ref[t]
    rq = bq // bk
    first = a == rq * qi + rq - 1
    overlaps_diag = a >= rq * qi

    q = q_ref[0]
    k = k_ref[0]
    v = v_ref[0]
    logits = jax.lax.dot_general(
        q, k, (((1,), (1,)), ((), ())), preferred_element_type=jnp.float32)
    log_om = jnp.minimum(-logits, 0.0) - jnp.log1p(jnp.exp(-jnp.abs(logits)))

    @pl.when(first)
    def _init():
        o_ref[...] = jnp.zeros_like(o_ref)
        carry_ref[...] = jnp.zeros_like(carry_ref)

    @pl.when(overlaps_diag)
    def _diag():
        i_g = qi * bq + jax.lax.broadcasted_iota(jnp.int32, (bq, bk), 0)
        j_g = a * bk + jax.lax.broadcasted_iota(jnp.int32, (bq, bk), 1)
        strict = j_g < i_g
        log_om_m = jnp.where(strict, log_om, 0.0)
        cs = _suffix_cumsum_incl(log_om_m, bk, split=True)
        carry = carry_ref[:, 0:1]
        att = jnp.where(strict, jnp.exp(logits + cs + carry), 0.0)
        o_ref[0] = o_ref[0] + jnp.dot(att.astype(jnp.bfloat16), v,
                                      preferred_element_type=jnp.float32)
        carry_ref[...] = carry_ref[...] + cs[:, 0:1]

    @pl.when(jnp.logical_not(overlaps_diag))
    def _off():
        cs = _suffix_cumsum_incl(log_om, bk, split=False)
        carry = carry_ref[:, 0:1]
        att = jnp.exp(logits + cs + carry)
        o_ref[0] = o_ref[0] + jnp.dot(att.astype(jnp.bfloat16), v,
                                      preferred_element_type=jnp.float32)
        carry_ref[...] = carry_ref[...] + cs[:, 0:1]


def _attention(qh, kh, vh, qi_of, a_of, nt, bq, bk):
    h, s, dh = qh.shape
    grid = (h, nt)

    def q_map(hh, t, qi_of_ref, a_of_ref):
        return (hh, qi_of_ref[t], 0)

    def kv_map(hh, t, qi_of_ref, a_of_ref):
        return (hh, a_of_ref[t], 0)

    grid_spec = pltpu.PrefetchScalarGridSpec(
        num_scalar_prefetch=2,
        grid=grid,
        in_specs=[
            pl.BlockSpec((1, bq, dh), q_map),
            pl.BlockSpec((1, bk, dh), kv_map),
            pl.BlockSpec((1, bk, dh), kv_map),
        ],
        out_specs=pl.BlockSpec((1, bq, dh), q_map),
        scratch_shapes=[pltpu.VMEM((bq, 128), jnp.float32)],
    )
    return pl.pallas_call(
        functools.partial(_attn_body, bq=bq, bk=bk),
        grid_spec=grid_spec,
        out_shape=jax.ShapeDtypeStruct((h, s, dh), jnp.float32),
        compiler_params=pltpu.CompilerParams(
            dimension_semantics=("parallel", "arbitrary")),
    )(qi_of, a_of, qh, kh, vh)


def _proj_body(attn_ref, res_ref, wo_ref, ln2_ref, wr_ref,
               h2_ref, x2_ref, comb_ref, f_ref, p_ref, *, n_exp):
    t = pl.program_id(0)
    h2 = jnp.dot(attn_ref[...].astype(jnp.bfloat16), wo_ref[...],
                 preferred_element_type=jnp.float32) + res_ref[...]
    h2_ref[...] = h2
    ms = jnp.mean(h2 * h2, axis=-1, keepdims=True)
    x2 = h2 * jax.lax.rsqrt(ms + EPS) * ln2_ref[...]
    x2_ref[...] = x2.astype(jnp.bfloat16)
    logits = jnp.dot(x2, wr_ref[...], preferred_element_type=jnp.float32)
    bt, ecols = logits.shape
    col = jax.lax.broadcasted_iota(jnp.int32, (bt, ecols), 1)
    valid = col < n_exp
    logits = jnp.where(valid, logits, NEG_BIG)
    m = jnp.max(logits, axis=-1, keepdims=True)
    p = jnp.exp(logits - m)
    p = p / jnp.sum(p, axis=-1, keepdims=True)
    pm = jnp.where(valid, p, -1.0)
    m1 = jnp.max(pm, axis=-1, keepdims=True)
    is1 = pm == m1
    pm2 = jnp.where(is1, -1.0, pm)
    m2 = jnp.max(pm2, axis=-1, keepdims=True)
    is2 = pm2 == m2
    denom = m1 + m2
    comb_ref[...] = (is1.astype(jnp.float32) * m1
                     + is2.astype(jnp.float32) * m2) / denom
    disp = is1.astype(jnp.float32) + is2.astype(jnp.float32)

    @pl.when(t == 0)
    def _init():
        f_ref[...] = jnp.zeros_like(f_ref)
        p_ref[...] = jnp.zeros_like(p_ref)

    f_ref[...] = f_ref[...] + jnp.sum(disp, axis=0, keepdims=True)
    p_ref[...] = p_ref[...] + jnp.sum(p, axis=0, keepdims=True)


def _proj_router(attn_flat, res, wo, ln2_w, wr_pad, n_exp, bt):
    s, d = attn_flat.shape
    ecols = wr_pad.shape[1]
    grid = (s // bt,)
    spec_row = pl.BlockSpec((bt, d), lambda t: (t, 0))
    return pl.pallas_call(
        functools.partial(_proj_body, n_exp=n_exp),
        grid=grid,
        in_specs=[
            spec_row,
            spec_row,
            pl.BlockSpec((d, d), lambda t: (0, 0)),
            pl.BlockSpec((1, d), lambda t: (0, 0)),
            pl.BlockSpec((d, ecols), lambda t: (0, 0)),
        ],
        out_specs=[
            spec_row,
            spec_row,
            pl.BlockSpec((bt, ecols), lambda t: (t, 0)),
            pl.BlockSpec((1, ecols), lambda t: (0, 0)),
            pl.BlockSpec((1, ecols), lambda t: (0, 0)),
        ],
        out_shape=[
            jax.ShapeDtypeStruct((s, d), jnp.float32),
            jax.ShapeDtypeStruct((s, d), jnp.bfloat16),
            jax.ShapeDtypeStruct((s, ecols), jnp.float32),
            jax.ShapeDtypeStruct((1, ecols), jnp.float32),
            jax.ShapeDtypeStruct((1, ecols), jnp.float32),
        ],
        compiler_params=pltpu.CompilerParams(dimension_semantics=("arbitrary",)),
    )(attn_flat, res, wo, ln2_w, wr_pad)


def _moe_body(x2_ref, h2_ref, w1_ref, w2_ref, c_ref, f_ref, p_ref,
              out_ref, aux_ref, *, n_exp, n_tok):
    t = pl.program_id(0)
    e = pl.program_id(1)

    @pl.when(e == 0)
    def _init():
        out_ref[...] = h2_ref[...]

    x = x2_ref[...]
    h = jnp.dot(x, w1_ref[0], preferred_element_type=jnp.float32)
    h = h * jax.nn.sigmoid(h)
    y = jnp.dot(h.astype(jnp.bfloat16), w2_ref[0],
                preferred_element_type=jnp.float32)
    out_ref[...] = out_ref[...] + y * c_ref[0]

    @pl.when(jnp.logical_and(t == 0, e == 0))
    def _aux():
        val = (jnp.float32(n_exp)
               * jnp.sum(f_ref[...] * p_ref[...]) / jnp.float32(n_tok * n_tok))
        aux_ref[...] = jnp.full_like(aux_ref, val)


def _moe(x2, h2, w1, w2, comb_ett, f_sum, p_sum, bt):
    s, d = x2.shape
    n_exp, _, dff = w1.shape
    ecols = f_sum.shape[1]
    grid = (s // bt, n_exp)
    spec_row = pl.BlockSpec((bt, d), lambda t, e: (t, 0))
    return pl.pallas_call(
        functools.partial(_moe_body, n_exp=n_exp, n_tok=s),
        grid=grid,
        in_specs=[
            spec_row,
            spec_row,
            pl.BlockSpec((1, d, dff), lambda t, e: (e, 0, 0)),
            pl.BlockSpec((1, dff, d), lambda t, e: (e, 0, 0)),
            pl.BlockSpec((1, bt, 1), lambda t, e: (e, t, 0)),
            pl.BlockSpec((1, ecols), lambda t, e: (0, 0)),
            pl.BlockSpec((1, ecols), lambda t, e: (0, 0)),
        ],
        out_specs=[
            spec_row,
            pl.BlockSpec((1, 128), lambda t, e: (0, 0)),
        ],
        out_shape=[
            jax.ShapeDtypeStruct((s, d), jnp.float32),
            jax.ShapeDtypeStruct((1, 128), jnp.float32),
        ],
        compiler_params=pltpu.CompilerParams(
            dimension_semantics=("parallel", "arbitrary")),
    )(x2, h2, w1, w2, comb_ett, f_sum, p_sum)


def _block_impl(hidden_states, ln1_w, wq, wk, wv, wo, ln2_w, w_router, w1, w2,
                n_heads):
    b, s, d = hidden_states.shape
    dh = d // n_heads
    n_exp = w_router.shape[1]
    bt = min(256, s)
    bq = min(1024, s)
    bk = min(256, s)

    x0 = hidden_states.reshape(s, d)
    q, k, v = _qkv(x0, ln1_w.reshape(1, d), wq, wk, wv, dh, bt)

    qh = q.reshape(s, n_heads, dh).transpose(1, 0, 2)
    kh = k.reshape(s, n_heads, dh).transpose(1, 0, 2)
    vh = v.reshape(s, n_heads, dh).transpose(1, 0, 2)

    nq = s // bq
    rq = bq // bk
    sched = [(qi, a) for qi in range(nq) for a in range(rq * qi + rq - 1, -1, -1)]
    qi_of = jnp.asarray([qi for qi, _ in sched], dtype=jnp.int32)
    a_of = jnp.asarray([a for _, a in sched], dtype=jnp.int32)
    attn = _attention(qh, kh, vh, qi_of, a_of, len(sched), bq, bk)
    attn_flat = attn.transpose(1, 0, 2).reshape(s, d)

    ecols = 128
    wr_pad = jnp.pad(w_router, ((0, 0), (0, ecols - n_exp)))
    h2, x2, comb, f_sum, p_sum = _proj_router(
        attn_flat, x0, wo.astype(jnp.bfloat16), ln2_w.reshape(1, d), wr_pad,
        n_exp, bt)

    comb_ett = comb[:, :n_exp].T[..., None]
    out, aux = _moe(x2, h2, w1.astype(jnp.bfloat16), w2.astype(jnp.bfloat16),
                    comb_ett, f_sum, p_sum, bt)
    return out.reshape(b, s, d), aux[0, 0]


def kernel(hidden_states, ln1_w, wq, wk, wv, wo, ln2_w, w_router, w1, w2):
    return _block_impl(hidden_states, ln1_w, wq, wk, wv, wo, ln2_w, w_router,
                       w1, w2, n_heads=16)


# speedup vs baseline: 1.0316x; 1.0316x over previous
import functools
import math

import jax
import jax.numpy as jnp
from jax.experimental import pallas as pl
from jax.experimental.pallas import tpu as pltpu

EPS = 1e-5
NEG_BIG = -1e30


def _log_sigmoid_pair(x):
    z = jnp.exp(-jnp.abs(x))
    l1p = jnp.log1p(z)
    return jnp.minimum(x, 0.0) - l1p, jnp.minimum(-x, 0.0) - l1p


def _qkv_body(x_ref, ln_ref, wq_ref, wk_ref, wv_ref, q_ref, k_ref, v_ref, *, scale):
    x = x_ref[...]
    ms = jnp.mean(x * x, axis=-1, keepdims=True)
    xn = x * jax.lax.rsqrt(ms + EPS) * ln_ref[...]
    q_ref[...] = jnp.dot(xn, wq_ref[...], preferred_element_type=jnp.float32) * scale
    k_ref[...] = jnp.dot(xn, wk_ref[...], preferred_element_type=jnp.float32)
    v_ref[...] = jnp.dot(xn.astype(jnp.bfloat16), wv_ref[...],
                         preferred_element_type=jnp.float32).astype(jnp.bfloat16)


def _qkv(x, ln1_w, wq, wk, wv, dh, bt):
    s, d = x.shape
    grid = (s // bt,)
    spec_row = pl.BlockSpec((bt, d), lambda t: (t, 0))
    spec_w = pl.BlockSpec((d, d), lambda t: (0, 0))
    spec_ln = pl.BlockSpec((1, d), lambda t: (0, 0))
    out = jax.ShapeDtypeStruct((s, d), jnp.float32)
    out_bf = jax.ShapeDtypeStruct((s, d), jnp.bfloat16)
    return pl.pallas_call(
        functools.partial(_qkv_body, scale=1.0 / math.sqrt(dh)),
        grid=grid,
        in_specs=[spec_row, spec_ln, spec_w, spec_w, spec_w],
        out_specs=[spec_row, spec_row, spec_row],
        out_shape=[out, out, out_bf],
        compiler_params=pltpu.CompilerParams(dimension_semantics=("parallel",)),
    )(x, ln1_w, wq, wk, wv.astype(jnp.bfloat16))


def _suffix_cumsum_incl(x, bk, split):
    tri = (jax.lax.broadcasted_iota(jnp.int32, (bk, bk), 0)
           >= jax.lax.broadcasted_iota(jnp.int32, (bk, bk), 1)).astype(jnp.bfloat16)
    hi = x.astype(jnp.bfloat16)
    cs = jnp.dot(hi, tri, preferred_element_type=jnp.float32)
    if split:
        lo = (x - hi.astype(jnp.float32)).astype(jnp.bfloat16)
        cs = cs + jnp.dot(lo, tri, preferred_element_type=jnp.float32)
    return cs


def _attn_body(qi_ref, a_ref, q_ref, k_ref, v_ref, o_ref, carry_ref, *, bq, bk):
    t = pl.program_id(1)
    qi = qi_ref[t]
    a = a_ref[t]
    rq = bq // bk
    first = a == rq * qi + rq - 1
    overlaps_diag = a >= rq * qi

    q = q_ref[0]
    k = k_ref[0]
    v = v_ref[0]
    logits = jax.lax.dot_general(
        q, k, (((1,), (1,)), ((), ())), preferred_element_type=jnp.float32)
    log_om = jnp.minimum(-logits, 0.0) - jnp.log1p(jnp.exp(-jnp.abs(logits)))

    @pl.when(first)
    def _init():
        o_ref[...] = jnp.zeros_like(o_ref)
        carry_ref[...] = jnp.zeros_like(carry_ref)

    @pl.when(overlaps_diag)
    def _diag():
        i_g = qi * bq + jax.lax.broadcasted_iota(jnp.int32, (bq, bk), 0)
        j_g = a * bk + jax.lax.broadcasted_iota(jnp.int32, (bq, bk), 1)
        strict = j_g < i_g
        log_om_m = jnp.where(strict, log_om, 0.0)
        cs = _suffix_cumsum_incl(log_om_m, bk, split=True)
        carry = carry_ref[:, 0:1]
        att = jnp.where(strict, jnp.exp(logits + cs + carry), 0.0)
        o_ref[0] = o_ref[0] + jnp.dot(att.astype(jnp.bfloat16), v,
                                      preferred_element_type=jnp.float32)
        carry_ref[...] = carry_ref[...] + cs[:, 0:1]

    @pl.when(jnp.logical_not(overlaps_diag))
    def _off():
        cs = _suffix_cumsum_incl(log_om, bk, split=False)
        carry = carry_ref[:, 0:1]
        att = jnp.exp(logits + cs + carry)
        o_ref[0] = o_ref[0] + jnp.dot(att.astype(jnp.bfloat16), v,
                                      preferred_element_type=jnp.float32)
        carry_ref[...] = carry_ref[...] + cs[:, 0:1]


def _attention(qh, kh, vh, qi_of, a_of, nt, bq, bk):
    h, s, dh = qh.shape
    grid = (h, nt)

    def q_map(hh, t, qi_of_ref, a_of_ref):
        return (hh, qi_of_ref[t], 0)

    def kv_map(hh, t, qi_of_ref, a_of_ref):
        return (hh, a_of_ref[t], 0)

    grid_spec = pltpu.PrefetchScalarGridSpec(
        num_scalar_prefetch=2,
        grid=grid,
        in_specs=[
            pl.BlockSpec((1, bq, dh), q_map),
            pl.BlockSpec((1, bk, dh), kv_map),
            pl.BlockSpec((1, bk, dh), kv_map),
        ],
        out_specs=pl.BlockSpec((1, bq, dh), q_map),
        scratch_shapes=[pltpu.VMEM((bq, 128), jnp.float32)],
    )
    return pl.pallas_call(
        functools.partial(_attn_body, bq=bq, bk=bk),
        grid_spec=grid_spec,
        out_shape=jax.ShapeDtypeStruct((h, s, dh), jnp.float32),
        compiler_params=pltpu.CompilerParams(
            dimension_semantics=("parallel", "arbitrary")),
    )(qi_of, a_of, qh, kh, vh)


def _proj_body(attn_ref, res_ref, wo_ref, ln2_ref, wr_ref,
               h2_ref, x2_ref, comb_ref, f_ref, p_ref, *, n_exp):
    t = pl.program_id(0)
    h2 = jnp.dot(attn_ref[...].astype(jnp.bfloat16), wo_ref[...],
                 preferred_element_type=jnp.float32) + res_ref[...]
    h2_ref[...] = h2
    ms = jnp.mean(h2 * h2, axis=-1, keepdims=True)
    x2 = h2 * jax.lax.rsqrt(ms + EPS) * ln2_ref[...]
    x2_ref[...] = x2.astype(jnp.bfloat16)
    logits = jnp.dot(x2, wr_ref[...], preferred_element_type=jnp.float32)
    bt, ecols = logits.shape
    col = jax.lax.broadcasted_iota(jnp.int32, (bt, ecols), 1)
    valid = col < n_exp
    logits = jnp.where(valid, logits, NEG_BIG)
    m = jnp.max(logits, axis=-1, keepdims=True)
    p = jnp.exp(logits - m)
    p = p / jnp.sum(p, axis=-1, keepdims=True)
    pm = jnp.where(valid, p, -1.0)
    m1 = jnp.max(pm, axis=-1, keepdims=True)
    is1 = pm == m1
    pm2 = jnp.where(is1, -1.0, pm)
    m2 = jnp.max(pm2, axis=-1, keepdims=True)
    is2 = pm2 == m2
    denom = m1 + m2
    comb_ref[...] = (is1.astype(jnp.float32) * m1
                     + is2.astype(jnp.float32) * m2) / denom
    disp = is1.astype(jnp.float32) + is2.astype(jnp.float32)

    @pl.when(t == 0)
    def _init():
        f_ref[...] = jnp.zeros_like(f_ref)
        p_ref[...] = jnp.zeros_like(p_ref)

    f_ref[...] = f_ref[...] + jnp.sum(disp, axis=0, keepdims=True)
    p_ref[...] = p_ref[...] + jnp.sum(p, axis=0, keepdims=True)


def _proj_router(attn_flat, res, wo, ln2_w, wr_pad, n_exp, bt):
    s, d = attn_flat.shape
    ecols = wr_pad.shape[1]
    grid = (s // bt,)
    spec_row = pl.BlockSpec((bt, d), lambda t: (t, 0))
    return pl.pallas_call(
        functools.partial(_proj_body, n_exp=n_exp),
        grid=grid,
        in_specs=[
            spec_row,
            spec_row,
            pl.BlockSpec((d, d), lambda t: (0, 0)),
            pl.BlockSpec((1, d), lambda t: (0, 0)),
            pl.BlockSpec((d, ecols), lambda t: (0, 0)),
        ],
        out_specs=[
            spec_row,
            spec_row,
            pl.BlockSpec((bt, ecols), lambda t: (t, 0)),
            pl.BlockSpec((1, ecols), lambda t: (0, 0)),
            pl.BlockSpec((1, ecols), lambda t: (0, 0)),
        ],
        out_shape=[
            jax.ShapeDtypeStruct((s, d), jnp.float32),
            jax.ShapeDtypeStruct((s, d), jnp.bfloat16),
            jax.ShapeDtypeStruct((s, ecols), jnp.float32),
            jax.ShapeDtypeStruct((1, ecols), jnp.float32),
            jax.ShapeDtypeStruct((1, ecols), jnp.float32),
        ],
        compiler_params=pltpu.CompilerParams(dimension_semantics=("arbitrary",)),
    )(attn_flat, res, wo, ln2_w, wr_pad)


def _moe_body(x2_ref, h2_ref, w1_ref, w2_ref, c_ref, f_ref, p_ref,
              out_ref, aux_ref, *, n_exp, n_tok):
    t = pl.program_id(0)
    e = pl.program_id(1)

    @pl.when(e == 0)
    def _init():
        out_ref[...] = h2_ref[...]

    x = x2_ref[...]
    h = jnp.dot(x, w1_ref[e], preferred_element_type=jnp.float32)
    h = h * jax.nn.sigmoid(h)
    y = jnp.dot(h.astype(jnp.bfloat16), w2_ref[e],
                preferred_element_type=jnp.float32)
    out_ref[...] = out_ref[...] + y * c_ref[0]

    @pl.when(jnp.logical_and(t == 0, e == 0))
    def _aux():
        val = (jnp.float32(n_exp)
               * jnp.sum(f_ref[...] * p_ref[...]) / jnp.float32(n_tok * n_tok))
        aux_ref[...] = jnp.full_like(aux_ref, val)


def _moe(x2, h2, w1, w2, comb_ett, f_sum, p_sum, bt):
    s, d = x2.shape
    n_exp, _, dff = w1.shape
    ecols = f_sum.shape[1]
    grid = (s // bt, n_exp)
    spec_row = pl.BlockSpec((bt, d), lambda t, e: (t, 0))
    return pl.pallas_call(
        functools.partial(_moe_body, n_exp=n_exp, n_tok=s),
        grid=grid,
        in_specs=[
            spec_row,
            spec_row,
            pl.BlockSpec((n_exp, d, dff), lambda t, e: (0, 0, 0)),
            pl.BlockSpec((n_exp, dff, d), lambda t, e: (0, 0, 0)),
            pl.BlockSpec((1, bt, 1), lambda t, e: (e, t, 0)),
            pl.BlockSpec((1, ecols), lambda t, e: (0, 0)),
            pl.BlockSpec((1, ecols), lambda t, e: (0, 0)),
        ],
        out_specs=[
            spec_row,
            pl.BlockSpec((1, 128), lambda t, e: (0, 0)),
        ],
        out_shape=[
            jax.ShapeDtypeStruct((s, d), jnp.float32),
            jax.ShapeDtypeStruct((1, 128), jnp.float32),
        ],
        compiler_params=pltpu.CompilerParams(
            dimension_semantics=("parallel", "arbitrary")),
    )(x2, h2, w1, w2, comb_ett, f_sum, p_sum)


def _block_impl(hidden_states, ln1_w, wq, wk, wv, wo, ln2_w, w_router, w1, w2,
                n_heads):
    b, s, d = hidden_states.shape
    dh = d // n_heads
    n_exp = w_router.shape[1]
    bt = min(256, s)
    bq = min(1024, s)
    bk = min(256, s)

    x0 = hidden_states.reshape(s, d)
    q, k, v = _qkv(x0, ln1_w.reshape(1, d), wq, wk, wv, dh, bt)

    qh = q.reshape(s, n_heads, dh).transpose(1, 0, 2)
    kh = k.reshape(s, n_heads, dh).transpose(1, 0, 2)
    vh = v.reshape(s, n_heads, dh).transpose(1, 0, 2)

    nq = s // bq
    rq = bq // bk
    sched = [(qi, a) for qi in range(nq) for a in range(rq * qi + rq - 1, -1, -1)]
    qi_of = jnp.asarray([qi for qi, _ in sched], dtype=jnp.int32)
    a_of = jnp.asarray([a for _, a in sched], dtype=jnp.int32)
    attn = _attention(qh, kh, vh, qi_of, a_of, len(sched), bq, bk)
    attn_flat = attn.transpose(1, 0, 2).reshape(s, d)

    ecols = 128
    wr_pad = jnp.pad(w_router, ((0, 0), (0, ecols - n_exp)))
    h2, x2, comb, f_sum, p_sum = _proj_router(
        attn_flat, x0, wo.astype(jnp.bfloat16), ln2_w.reshape(1, d), wr_pad,
        n_exp, bt)

    comb_ett = comb[:, :n_exp].T[..., None]
    out, aux = _moe(x2, h2, w1.astype(jnp.bfloat16), w2.astype(jnp.bfloat16),
                    comb_ett, f_sum, p_sum, bt)
    return out.reshape(b, s, d), aux[0, 0]


def kernel(hidden_states, ln1_w, wq, wk, wv, wo, ln2_w, w_router, w1, w2):
    return _block_impl(hidden_states, ln1_w, wq, wk, wv, wo, ln2_w, w_router,
                       w1, w2, n_heads=16)


# speedup vs baseline: 1.1109x; 1.0769x over previous
import functools
import math

import jax
import jax.numpy as jnp
from jax.experimental import pallas as pl
from jax.experimental.pallas import tpu as pltpu

EPS = 1e-5
NEG_BIG = -1e30


def _log2_1p(z):
    return jnp.log2(1.0 + z)


def _qkv_body(x_ref, ln_ref, wq_ref, wk_ref, wv_ref, q_ref, k_ref, v_ref, *, scale):
    x = x_ref[...]
    ms = jnp.mean(x * x, axis=-1, keepdims=True)
    xn = x * jax.lax.rsqrt(ms + EPS) * ln_ref[...]
    q_ref[...] = jnp.dot(xn, wq_ref[...], preferred_element_type=jnp.float32) * scale
    k_ref[...] = jnp.dot(xn, wk_ref[...], preferred_element_type=jnp.float32)
    v_ref[...] = jnp.dot(xn.astype(jnp.bfloat16), wv_ref[...],
                         preferred_element_type=jnp.float32).astype(jnp.bfloat16)


def _qkv(x, ln1_w, wq, wk, wv, dh, bt):
    s, d = x.shape
    grid = (s // bt,)
    spec_row = pl.BlockSpec((bt, d), lambda t: (t, 0))
    spec_w = pl.BlockSpec((d, d), lambda t: (0, 0))
    spec_ln = pl.BlockSpec((1, d), lambda t: (0, 0))
    out = jax.ShapeDtypeStruct((s, d), jnp.float32)
    out_bf = jax.ShapeDtypeStruct((s, d), jnp.bfloat16)
    return pl.pallas_call(
        functools.partial(_qkv_body, scale=math.log2(math.e) / math.sqrt(dh)),
        grid=grid,
        in_specs=[spec_row, spec_ln, spec_w, spec_w, spec_w],
        out_specs=[spec_row, spec_row, spec_row],
        out_shape=[out, out, out_bf],
        compiler_params=pltpu.CompilerParams(dimension_semantics=("parallel",)),
    )(x, ln1_w, wq, wk, wv.astype(jnp.bfloat16))


def _suffix_cumsum_incl(x, bk, split):
    tri = (jax.lax.broadcasted_iota(jnp.int32, (bk, bk), 0)
           >= jax.lax.broadcasted_iota(jnp.int32, (bk, bk), 1)).astype(jnp.bfloat16)
    hi = x.astype(jnp.bfloat16)
    cs = jnp.dot(hi, tri, preferred_element_type=jnp.float32)
    if split:
        lo = (x - hi.astype(jnp.float32)).astype(jnp.bfloat16)
        cs = cs + jnp.dot(lo, tri, preferred_element_type=jnp.float32)
    return cs


def _attn_body(qi_ref, a_ref, q_ref, k_ref, v_ref, o_ref, carry_ref, *, bq, bk):
    t = pl.program_id(1)
    qi = qi_ref[t]
    a = a_ref[t]
    rq = bq // bk
    first = a == rq * qi + rq - 1
    overlaps_diag = a >= rq * qi

    q = q_ref[0]
    k = k_ref[0]
    v = v_ref[0]
    y = jax.lax.dot_general(
        q, k, (((1,), (1,)), ((), ())), preferred_element_type=jnp.float32)
    log_om = jnp.minimum(-y, 0.0) - _log2_1p(jnp.exp2(-jnp.abs(y)))

    @pl.when(first)
    def _init():
        o_ref[...] = jnp.zeros_like(o_ref)
        carry_ref[...] = jnp.zeros_like(carry_ref)

    @pl.when(overlaps_diag)
    def _diag():
        i_g = qi * bq + jax.lax.broadcasted_iota(jnp.int32, (bq, bk), 0)
        j_g = a * bk + jax.lax.broadcasted_iota(jnp.int32, (bq, bk), 1)
        strict = j_g < i_g
        log_om_m = jnp.where(strict, log_om, 0.0)
        cs = _suffix_cumsum_incl(log_om_m, bk, split=True)
        carry = carry_ref[:, 0:1]
        att = jnp.where(strict, jnp.exp2(y + cs + carry), 0.0)
        o_ref[0] = o_ref[0] + jnp.dot(att.astype(jnp.bfloat16), v,
                                      preferred_element_type=jnp.float32)
        carry_ref[...] = carry_ref[...] + cs[:, 0:1]

    @pl.when(jnp.logical_not(overlaps_diag))
    def _off():
        cs = _suffix_cumsum_incl(log_om, bk, split=False)
        carry = carry_ref[:, 0:1]
        att = jnp.exp2(y + cs + carry)
        o_ref[0] = o_ref[0] + jnp.dot(att.astype(jnp.bfloat16), v,
                                      preferred_element_type=jnp.float32)
        carry_ref[...] = carry_ref[...] + cs[:, 0:1]


def _attention(qh, kh, vh, qi_of, a_of, nt, bq, bk):
    h, s, dh = qh.shape
    grid = (h, nt)

    def q_map(hh, t, qi_of_ref, a_of_ref):
        return (hh, qi_of_ref[t], 0)

    def kv_map(hh, t, qi_of_ref, a_of_ref):
        return (hh, a_of_ref[t], 0)

    grid_spec = pltpu.PrefetchScalarGridSpec(
        num_scalar_prefetch=2,
        grid=grid,
        in_specs=[
            pl.BlockSpec((1, bq, dh), q_map),
            pl.BlockSpec((1, bk, dh), kv_map),
            pl.BlockSpec((1, bk, dh), kv_map),
        ],
        out_specs=pl.BlockSpec((1, bq, dh), q_map),
        scratch_shapes=[pltpu.VMEM((bq, 128), jnp.float32)],
    )
    return pl.pallas_call(
        functools.partial(_attn_body, bq=bq, bk=bk),
        grid_spec=grid_spec,
        out_shape=jax.ShapeDtypeStruct((h, s, dh), jnp.float32),
        compiler_params=pltpu.CompilerParams(
            dimension_semantics=("parallel", "arbitrary")),
    )(qi_of, a_of, qh, kh, vh)


def _proj_body(attn_ref, res_ref, wo_ref, ln2_ref, wr_ref,
               h2_ref, x2_ref, comb_ref, f_ref, p_ref, *, n_exp):
    t = pl.program_id(0)
    h2 = jnp.dot(attn_ref[...].astype(jnp.bfloat16), wo_ref[...],
                 preferred_element_type=jnp.float32) + res_ref[...]
    h2_ref[...] = h2
    ms = jnp.mean(h2 * h2, axis=-1, keepdims=True)
    x2 = h2 * jax.lax.rsqrt(ms + EPS) * ln2_ref[...]
    x2_ref[...] = x2.astype(jnp.bfloat16)
    logits = jnp.dot(x2, wr_ref[...], preferred_element_type=jnp.float32)
    bt, ecols = logits.shape
    col = jax.lax.broadcasted_iota(jnp.int32, (bt, ecols), 1)
    valid = col < n_exp
    logits = jnp.where(valid, logits, NEG_BIG)
    m = jnp.max(logits, axis=-1, keepdims=True)
    p = jnp.exp(logits - m)
    p = p / jnp.sum(p, axis=-1, keepdims=True)
    pm = jnp.where(valid, p, -1.0)
    m1 = jnp.max(pm, axis=-1, keepdims=True)
    is1 = pm == m1
    pm2 = jnp.where(is1, -1.0, pm)
    m2 = jnp.max(pm2, axis=-1, keepdims=True)
    is2 = pm2 == m2
    denom = m1 + m2
    comb_ref[...] = (is1.astype(jnp.float32) * m1
                     + is2.astype(jnp.float32) * m2) / denom
    disp = is1.astype(jnp.float32) + is2.astype(jnp.float32)

    @pl.when(t == 0)
    def _init():
        f_ref[...] = jnp.zeros_like(f_ref)
        p_ref[...] = jnp.zeros_like(p_ref)

    f_ref[...] = f_ref[...] + jnp.sum(disp, axis=0, keepdims=True)
    p_ref[...] = p_ref[...] + jnp.sum(p, axis=0, keepdims=True)


def _proj_router(attn_flat, res, wo, ln2_w, wr_pad, n_exp, bt):
    s, d = attn_flat.shape
    ecols = wr_pad.shape[1]
    grid = (s // bt,)
    spec_row = pl.BlockSpec((bt, d), lambda t: (t, 0))
    return pl.pallas_call(
        functools.partial(_proj_body, n_exp=n_exp),
        grid=grid,
        in_specs=[
            spec_row,
            spec_row,
            pl.BlockSpec((d, d), lambda t: (0, 0)),
            pl.BlockSpec((1, d), lambda t: (0, 0)),
            pl.BlockSpec((d, ecols), lambda t: (0, 0)),
        ],
        out_specs=[
            spec_row,
            spec_row,
            pl.BlockSpec((bt, ecols), lambda t: (t, 0)),
            pl.BlockSpec((1, ecols), lambda t: (0, 0)),
            pl.BlockSpec((1, ecols), lambda t: (0, 0)),
        ],
        out_shape=[
            jax.ShapeDtypeStruct((s, d), jnp.float32),
            jax.ShapeDtypeStruct((s, d), jnp.bfloat16),
            jax.ShapeDtypeStruct((s, ecols), jnp.float32),
            jax.ShapeDtypeStruct((1, ecols), jnp.float32),
            jax.ShapeDtypeStruct((1, ecols), jnp.float32),
        ],
        compiler_params=pltpu.CompilerParams(dimension_semantics=("arbitrary",)),
    )(attn_flat, res, wo, ln2_w, wr_pad)


def _moe_body(x2_ref, h2_ref, w1_ref, w2_ref, c_ref, f_ref, p_ref,
              out_ref, aux_ref, *, n_exp, n_tok):
    t = pl.program_id(0)
    e = pl.program_id(1)

    @pl.when(e == 0)
    def _init():
        out_ref[...] = h2_ref[...]

    x = x2_ref[...]
    h = jnp.dot(x, w1_ref[e], preferred_element_type=jnp.float32)
    h = h * jax.nn.sigmoid(h)
    y = jnp.dot(h.astype(jnp.bfloat16), w2_ref[e],
                preferred_element_type=jnp.float32)
    out_ref[...] = out_ref[...] + y * c_ref[0]

    @pl.when(jnp.logical_and(t == 0, e == 0))
    def _aux():
        val = (jnp.float32(n_exp)
               * jnp.sum(f_ref[...] * p_ref[...]) / jnp.float32(n_tok * n_tok))
        aux_ref[...] = jnp.full_like(aux_ref, val)


def _moe(x2, h2, w1, w2, comb_ett, f_sum, p_sum, bt):
    s, d = x2.shape
    n_exp, _, dff = w1.shape
    ecols = f_sum.shape[1]
    grid = (s // bt, n_exp)
    spec_row = pl.BlockSpec((bt, d), lambda t, e: (t, 0))
    return pl.pallas_call(
        functools.partial(_moe_body, n_exp=n_exp, n_tok=s),
        grid=grid,
        in_specs=[
            spec_row,
            spec_row,
            pl.BlockSpec((n_exp, d, dff), lambda t, e: (0, 0, 0)),
            pl.BlockSpec((n_exp, dff, d), lambda t, e: (0, 0, 0)),
            pl.BlockSpec((1, bt, 1), lambda t, e: (e, t, 0)),
            pl.BlockSpec((1, ecols), lambda t, e: (0, 0)),
            pl.BlockSpec((1, ecols), lambda t, e: (0, 0)),
        ],
        out_specs=[
            spec_row,
            pl.BlockSpec((1, 128), lambda t, e: (0, 0)),
        ],
        out_shape=[
            jax.ShapeDtypeStruct((s, d), jnp.float32),
            jax.ShapeDtypeStruct((1, 128), jnp.float32),
        ],
        compiler_params=pltpu.CompilerParams(
            dimension_semantics=("parallel", "arbitrary")),
    )(x2, h2, w1, w2, comb_ett, f_sum, p_sum)


def _block_impl(hidden_states, ln1_w, wq, wk, wv, wo, ln2_w, w_router, w1, w2,
                n_heads):
    b, s, d = hidden_states.shape
    dh = d // n_heads
    n_exp = w_router.shape[1]
    bt = min(256, s)
    bq = min(1024, s)
    bk = min(256, s)

    x0 = hidden_states.reshape(s, d)
    q, k, v = _qkv(x0, ln1_w.reshape(1, d), wq, wk, wv, dh, bt)

    qh = q.reshape(s, n_heads, dh).transpose(1, 0, 2)
    kh = k.reshape(s, n_heads, dh).transpose(1, 0, 2)
    vh = v.reshape(s, n_heads, dh).transpose(1, 0, 2)

    nq = s // bq
    rq = bq // bk
    sched = [(qi, a) for qi in range(nq) for a in range(rq * qi + rq - 1, -1, -1)]
    qi_of = jnp.asarray([qi for qi, _ in sched], dtype=jnp.int32)
    a_of = jnp.asarray([a for _, a in sched], dtype=jnp.int32)
    attn = _attention(qh, kh, vh, qi_of, a_of, len(sched), bq, bk)
    attn_flat = attn.transpose(1, 0, 2).reshape(s, d)

    ecols = 128
    wr_pad = jnp.pad(w_router, ((0, 0), (0, ecols - n_exp)))
    h2, x2, comb, f_sum, p_sum = _proj_router(
        attn_flat, x0, wo.astype(jnp.bfloat16), ln2_w.reshape(1, d), wr_pad,
        n_exp, bt)

    comb_ett = comb[:, :n_exp].T[..., None]
    out, aux = _moe(x2, h2, w1.astype(jnp.bfloat16), w2.astype(jnp.bfloat16),
                    comb_ett, f_sum, p_sum, bt)
    return out.reshape(b, s, d), aux[0, 0]


def kernel(hidden_states, ln1_w, wq, wk, wv, wo, ln2_w, w_router, w1, w2):
    return _block_impl(hidden_states, ln1_w, wq, wk, wv, wo, ln2_w, w_router,
                       w1, w2, n_heads=16)
